# baseline (speedup 1.0000x reference)
import functools

import jax
import jax.numpy as jnp
from jax import lax
from jax.experimental import pallas as pl
from jax.experimental.pallas import tpu as pltpu
from jax.experimental.pallas import tpu_sc as plsc

N = 100000
E = 3200000
D_IN = 10
D_HID = 16
D_OUT = 2

NC = 2
NS = 16
NW = NC * NS
BLK = 128
BPW = 800
KB = 16
CHUNKS = BPW // KB
KBA = 10
CHUNKS_A = BPW // KBA
NB = NW * BPW
BPW0_D = 992
BPW1_D = NB // NS - BPW0_D
BPW0_A = 1600
BPW1_A = NB // NS - BPW0_A
EP = NB * BLK
NP = 100352
RPS = NP // NS

def _deg_body(src_h, dst_h, out_h, idx_s, idx_d, ones_v, zb, acc_s, acc_d,
              sem_i, sem_s):
    c = lax.axis_index("c")
    s = lax.axis_index("s")
    wid = c * NS + s

    def fill(i, _):
        zb[pl.ds(i * 16, 16)] = jnp.zeros((16,), jnp.float32)
        return 0

    lax.fori_loop(0, RPS // 16, fill, 0)
    for i in range(BLK // 16):
        ones_v[pl.ds(i * 16, 16)] = jnp.ones((16,), jnp.float32)
    pltpu.sync_copy(zb, acc_s.at[pl.ds(s * RPS, RPS)])
    pltpu.sync_copy(zb, acc_d.at[pl.ds(s * RPS, RPS)])
    plsc.subcore_barrier()

    base = lax.select(c == 0, s * BPW0_D, NS * BPW0_D + s * BPW1_D)
    nchunks = lax.select(c == 0, BPW0_D // KB, BPW1_D // KB)

    def load_idx(k):
        blk0 = base + k * KB
        par = lax.rem(k, 2)
        pltpu.async_copy(src_h.at[pl.ds(blk0, KB)], idx_s.at[par], sem_i)
        pltpu.async_copy(dst_h.at[pl.ds(blk0, KB)], idx_d.at[par], sem_i)

    load_idx(0)

    def chunk(k, _):
        par = lax.rem(k, 2)
        pltpu.make_async_copy(src_h.at[pl.ds(0, KB)], idx_s.at[par], sem_i).wait()
        pltpu.make_async_copy(dst_h.at[pl.ds(0, KB)], idx_d.at[par], sem_i).wait()

        @pl.when(k < nchunks - 1)
        def _():
            load_idx(k + 1)

        for j in range(KB):
            pltpu.sync_copy(ones_v, acc_s.at[idx_s.at[par, j]], add=True)
            pltpu.sync_copy(ones_v, acc_d.at[idx_d.at[par, j]], add=True)
        return 0

    lax.fori_loop(0, nchunks, chunk, 0)
    plsc.subcore_barrier()
    pltpu.sync_copy(acc_s.at[pl.ds(s * RPS, RPS)], out_h.at[c, 0, pl.ds(s * RPS, RPS)])
    pltpu.sync_copy(acc_d.at[pl.ds(s * RPS, RPS)], out_h.at[c, 1, pl.ds(s * RPS, RPS)])


@functools.lru_cache(maxsize=None)
def _build_deg():
    return pl.kernel(
        _deg_body,
        out_type=jax.ShapeDtypeStruct((NC, 2, NP), jnp.float32),
        mesh=plsc.VectorSubcoreMesh(core_axis_name="c", subcore_axis_name="s"),
        compiler_params=pltpu.CompilerParams(
            use_tc_tiling_on_sc=False, skip_device_barrier=True),
        scratch_types=[
            pltpu.VMEM((2, KB, BLK), jnp.int32),
            pltpu.VMEM((2, KB, BLK), jnp.int32),
            pltpu.VMEM((BLK,), jnp.float32),
            pltpu.VMEM((RPS,), jnp.float32),
            pltpu.VMEM_SHARED((NP,), jnp.float32),
            pltpu.VMEM_SHARED((NP,), jnp.float32),
            pltpu.SemaphoreType.DMA,
            pltpu.SemaphoreType.DMA,
        ],
    )


def _agg_body(src_h, dst_h, tab_h, out_h, idx_s, idx_d, rows, zb, acc,
              sem_i, sem_g, sem_s):
    c = lax.axis_index("c")
    s = lax.axis_index("s")
    wid = c * NS + s

    def fillz(i, _):
        zb[i] = jnp.zeros((16,), jnp.float32)
        return 0

    lax.fori_loop(0, BLK, fillz, 0)

    def zero_acc(i, _):
        pltpu.sync_copy(zb, acc.at[pl.ds(s * RPS + i * BLK, BLK)])
        return 0

    lax.fori_loop(0, RPS // BLK, zero_acc, 0)
    plsc.subcore_barrier()

    base = lax.select(c == 0, s * BPW0_A, NS * BPW0_A + s * BPW1_A)
    base = lax.min(base, NB - KBA)
    nchunks = lax.select(c == 0, BPW0_A // KBA, BPW1_A // KBA)

    def load_idx(k):
        blk0 = base + k * KBA
        par = lax.rem(k, 2)
        pltpu.async_copy(src_h.at[pl.ds(blk0, KBA)], idx_s.at[par], sem_i)
        pltpu.async_copy(dst_h.at[pl.ds(blk0, KBA)], idx_d.at[par], sem_i)

    @pl.when(nchunks > 0)
    def _():
        load_idx(0)

    def chunk(k, _):
        par = lax.rem(k, 2)
        @pl.when(k > 0)
        def _():
            pltpu.make_async_copy(
                tab_h.at[pl.ds(0, KBA * BLK)], rows, sem_s).wait()

        pltpu.make_async_copy(src_h.at[pl.ds(0, KBA)], idx_s.at[par], sem_i).wait()
        pltpu.make_async_copy(dst_h.at[pl.ds(0, KBA)], idx_d.at[par], sem_i).wait()

        @pl.when(k < nchunks - 1)
        def _():
            load_idx(k + 1)

        hs = []
        for j in range(KBA):
            hs.append(pltpu.async_copy(
                tab_h.at[idx_s.at[par, j]],
                rows.at[pl.ds(j * BLK, BLK)], sem_g))
        for j in range(KBA):
            hs[j].wait()
            pltpu.async_copy(rows.at[pl.ds(j * BLK, BLK)],
                             acc.at[idx_d.at[par, j]], sem_s, add=True)
        return 0

    lax.fori_loop(0, nchunks, chunk, 0)

    @pl.when(nchunks > 0)
    def _():
        pltpu.make_async_copy(tab_h.at[pl.ds(0, KBA * BLK)], rows, sem_s).wait()

    plsc.subcore_barrier()
    pltpu.sync_copy(acc.at[pl.ds(s * RPS, RPS)], out_h.at[c, pl.ds(s * RPS, RPS)])


@functools.lru_cache(maxsize=None)
def _build_agg():
    return pl.kernel(
        _agg_body,
        out_type=jax.ShapeDtypeStruct((NC, NP, D_HID), jnp.float32),
        mesh=plsc.VectorSubcoreMesh(core_axis_name="c", subcore_axis_name="s"),
        compiler_params=pltpu.CompilerParams(
            use_tc_tiling_on_sc=False, skip_device_barrier=True),
        scratch_types=[
            pltpu.VMEM((2, KBA, BLK), jnp.int32),
            pltpu.VMEM((2, KBA, BLK), jnp.int32),
            pltpu.VMEM((KBA * BLK, D_HID), jnp.float32),
            pltpu.VMEM((BLK, D_HID), jnp.float32),
            pltpu.VMEM_SHARED((NP, D_HID), jnp.float32),
            pltpu.SemaphoreType.DMA,
            pltpu.SemaphoreType.DMA,
            pltpu.SemaphoreType.DMA,
        ],
    )


def _deg_pass(srcp, dstp):
    return _build_deg()(srcp, dstp)


def _agg_pass(srcp, dstp, table):
    return _build_agg()(srcp, dstp, table)


BN = 1024


def _norm(d):
    return lax.rsqrt(jnp.maximum(d, 1.0)).reshape(BN, 1)


def _tc0_body(f_ref, w_ref, o_ref):
    o_ref[...] = jnp.dot(f_ref[...], w_ref[...],
                         preferred_element_type=jnp.float32)


def _tc1_body(h_ref, dg_ref, o_ref):
    ns = _norm(dg_ref[0, 0] + dg_ref[1, 0])
    o_ref[...] = h_ref[...] * ns


def _tc2_body(m_ref, dg_ref, b_ref, o_ref):
    m = m_ref[0] + m_ref[1]
    nd = _norm(dg_ref[0, 1] + dg_ref[1, 1])
    ns = _norm(dg_ref[0, 0] + dg_ref[1, 0])
    x = jnp.maximum(m * nd + b_ref[...], 0.0)
    o_ref[...] = x * ns


def _tc3_body(m_ref, dg_ref, w_ref, b_ref, o_ref):
    m = m_ref[0] + m_ref[1]
    nd = _norm(dg_ref[0, 1] + dg_ref[1, 1])
    o_ref[...] = (jnp.dot(m, w_ref[...], preferred_element_type=jnp.float32)
                  * nd + b_ref[...])[:, :D_OUT]


_deg_spec = pl.BlockSpec((NC, 2, BN), lambda i: (0, 0, i))
_vec_spec = pl.BlockSpec((BN, D_HID), lambda i: (i, 0))
_par_spec = pl.BlockSpec((NC, BN, D_HID), lambda i: (0, i, 0))
_b_spec = pl.BlockSpec((1, D_HID), lambda i: (0, 0))

_tc0_call = pl.pallas_call(
    _tc0_body,
    grid=(NP // BN,),
    in_specs=[pl.BlockSpec((BN, D_IN), lambda i: (i, 0)),
              pl.BlockSpec((D_IN, D_HID), lambda i: (0, 0))],
    out_specs=_vec_spec,
    out_shape=jax.ShapeDtypeStruct((NP, D_HID), jnp.float32),
)

_tc1_call = pl.pallas_call(
    _tc1_body,
    grid=(NP // BN,),
    in_specs=[_vec_spec, _deg_spec],
    out_specs=_vec_spec,
    out_shape=jax.ShapeDtypeStruct((NP, D_HID), jnp.float32),
)

_tc2_call = pl.pallas_call(
    _tc2_body,
    grid=(NP // BN,),
    in_specs=[_par_spec, _deg_spec, _b_spec],
    out_specs=_vec_spec,
    out_shape=jax.ShapeDtypeStruct((NP, D_HID), jnp.float32),
)

_tc3_call = pl.pallas_call(
    _tc3_body,
    grid=(NP // BN,),
    in_specs=[_par_spec, _deg_spec,
              pl.BlockSpec((D_HID, D_HID), lambda i: (0, 0)), _b_spec],
    out_specs=pl.BlockSpec((BN, D_OUT), lambda i: (i, 0)),
    out_shape=jax.ShapeDtypeStruct((N, D_OUT), jnp.float32),
)


def kernel(features, edge_index, W1, b1, W2, b2):
    src = edge_index[0]
    dst = edge_index[1]
    pad = jnp.full((EP - E,), N, dtype=jnp.int32)
    srcp = jnp.concatenate([src, pad]).reshape(NB, BLK)
    dstp = jnp.concatenate([dst, pad]).reshape(NB, BLK)
    W2p = jnp.pad(W2, ((0, 0), (0, D_HID - D_OUT)))
    b1r = b1.reshape(1, D_HID)
    b2p = jnp.pad(b2, (0, D_HID - D_OUT)).reshape(1, D_HID)

    H = _tc0_call(features, W1)
    degs = _deg_pass(srcp, dstp)
    T1 = _tc1_call(H, degs)
    M1 = _agg_pass(srcp, dstp, T1)
    T2 = _tc2_call(M1, degs, b1r)
    M2 = _agg_pass(srcp, dstp, T2)
    return _tc3_call(M2, degs, W2p, b2p)


# speedup vs baseline: 1.2336x; 1.2336x over previous
import functools

import jax
import jax.numpy as jnp
from jax import lax
from jax.experimental import pallas as pl
from jax.experimental.pallas import tpu as pltpu
from jax.experimental.pallas import tpu_sc as plsc

N = 100000
E = 3200000
D_IN = 10
D_HID = 16
D_OUT = 2

NC = 2
NS = 16
NW = NC * NS
BLK = 128
BPW = 800
KB = 16
CHUNKS = BPW // KB
KBA = 10
CHUNKS_A = BPW // KBA
NB = NW * BPW
BPW0_D = 992
BPW1_D = NB // NS - BPW0_D
BPW0_A = 1400
BPW1_A = NB // NS - BPW0_A
EP = NB * BLK
NP = 100352
RPS = NP // NS

def _deg_body(src_h, dst_h, out_h, idx_s, idx_d, ones_v, zb, acc_s, acc_d,
              sem_i, sem_s):
    c = lax.axis_index("c")
    s = lax.axis_index("s")
    wid = c * NS + s

    def fill(i, _):
        zb[pl.ds(i * 16, 16)] = jnp.zeros((16,), jnp.float32)
        return 0

    lax.fori_loop(0, RPS // 16, fill, 0)
    for i in range(BLK // 16):
        ones_v[pl.ds(i * 16, 16)] = jnp.ones((16,), jnp.float32)
    pltpu.sync_copy(zb, acc_s.at[pl.ds(s * RPS, RPS)])
    pltpu.sync_copy(zb, acc_d.at[pl.ds(s * RPS, RPS)])
    plsc.subcore_barrier()

    base = lax.select(c == 0, s * BPW0_D, NS * BPW0_D + s * BPW1_D)
    nchunks = lax.select(c == 0, BPW0_D // KB, BPW1_D // KB)

    def load_idx(k):
        blk0 = base + k * KB
        par = lax.rem(k, 2)
        pltpu.async_copy(src_h.at[pl.ds(blk0, KB)], idx_s.at[par], sem_i)
        pltpu.async_copy(dst_h.at[pl.ds(blk0, KB)], idx_d.at[par], sem_i)

    load_idx(0)

    def chunk(k, _):
        par = lax.rem(k, 2)
        pltpu.make_async_copy(src_h.at[pl.ds(0, KB)], idx_s.at[par], sem_i).wait()
        pltpu.make_async_copy(dst_h.at[pl.ds(0, KB)], idx_d.at[par], sem_i).wait()

        @pl.when(k < nchunks - 1)
        def _():
            load_idx(k + 1)

        for j in range(KB):
            pltpu.sync_copy(ones_v, acc_s.at[idx_s.at[par, j]], add=True)
            pltpu.sync_copy(ones_v, acc_d.at[idx_d.at[par, j]], add=True)
        return 0

    lax.fori_loop(0, nchunks, chunk, 0)
    plsc.subcore_barrier()
    pltpu.sync_copy(acc_s.at[pl.ds(s * RPS, RPS)], out_h.at[c, 0, pl.ds(s * RPS, RPS)])
    pltpu.sync_copy(acc_d.at[pl.ds(s * RPS, RPS)], out_h.at[c, 1, pl.ds(s * RPS, RPS)])


@functools.lru_cache(maxsize=None)
def _build_deg():
    return pl.kernel(
        _deg_body,
        out_type=jax.ShapeDtypeStruct((NC, 2, NP), jnp.float32),
        mesh=plsc.VectorSubcoreMesh(core_axis_name="c", subcore_axis_name="s"),
        compiler_params=pltpu.CompilerParams(
            use_tc_tiling_on_sc=False, skip_device_barrier=True),
        scratch_types=[
            pltpu.VMEM((2, KB, BLK), jnp.int32),
            pltpu.VMEM((2, KB, BLK), jnp.int32),
            pltpu.VMEM((BLK,), jnp.float32),
            pltpu.VMEM((RPS,), jnp.float32),
            pltpu.VMEM_SHARED((NP,), jnp.float32),
            pltpu.VMEM_SHARED((NP,), jnp.float32),
            pltpu.SemaphoreType.DMA,
            pltpu.SemaphoreType.DMA,
        ],
    )


def _agg_body(src_h, dst_h, tab_h, out_h, idx_s, idx_d, rows, zb, acc,
              sem_i, sem_g, sem_s):
    c = lax.axis_index("c")
    s = lax.axis_index("s")
    wid = c * NS + s

    def fillz(i, _):
        zb[i] = jnp.zeros((16,), jnp.float32)
        return 0

    lax.fori_loop(0, BLK, fillz, 0)

    def zero_acc(i, _):
        pltpu.sync_copy(zb, acc.at[pl.ds(s * RPS + i * BLK, BLK)])
        return 0

    lax.fori_loop(0, RPS // BLK, zero_acc, 0)
    plsc.subcore_barrier()

    base = lax.select(c == 0, s * BPW0_A, NS * BPW0_A + s * BPW1_A)
    base = lax.min(base, NB - KBA)
    nchunks = lax.select(c == 0, BPW0_A // KBA, BPW1_A // KBA)

    def load_idx(k):
        blk0 = base + k * KBA
        par = lax.rem(k, 2)
        pltpu.async_copy(src_h.at[pl.ds(blk0, KBA)], idx_s.at[par], sem_i)
        pltpu.async_copy(dst_h.at[pl.ds(blk0, KBA)], idx_d.at[par], sem_i)

    @pl.when(nchunks > 0)
    def _():
        load_idx(0)

    def chunk(k, _):
        par = lax.rem(k, 2)
        @pl.when(k > 0)
        def _():
            pltpu.make_async_copy(
                tab_h.at[pl.ds(0, KBA * BLK)], rows, sem_s).wait()

        pltpu.make_async_copy(src_h.at[pl.ds(0, KBA)], idx_s.at[par], sem_i).wait()
        pltpu.make_async_copy(dst_h.at[pl.ds(0, KBA)], idx_d.at[par], sem_i).wait()

        @pl.when(k < nchunks - 1)
        def _():
            load_idx(k + 1)

        hs = []
        for j in range(KBA):
            hs.append(pltpu.async_copy(
                tab_h.at[idx_s.at[par, j]],
                rows.at[pl.ds(j * BLK, BLK)], sem_g))
        for j in range(KBA):
            hs[j].wait()
            pltpu.async_copy(rows.at[pl.ds(j * BLK, BLK)],
                             acc.at[idx_d.at[par, j]], sem_s, add=True)
        return 0

    lax.fori_loop(0, nchunks, chunk, 0)

    @pl.when(nchunks > 0)
    def _():
        pltpu.make_async_copy(tab_h.at[pl.ds(0, KBA * BLK)], rows, sem_s).wait()

    plsc.subcore_barrier()
    pltpu.sync_copy(acc.at[pl.ds(s * RPS, RPS)], out_h.at[c, pl.ds(s * RPS, RPS)])


@functools.lru_cache(maxsize=None)
def _build_agg():
    return pl.kernel(
        _agg_body,
        out_type=jax.ShapeDtypeStruct((NC, NP, D_HID), jnp.float32),
        mesh=plsc.VectorSubcoreMesh(core_axis_name="c", subcore_axis_name="s"),
        compiler_params=pltpu.CompilerParams(
            use_tc_tiling_on_sc=False, skip_device_barrier=True),
        scratch_types=[
            pltpu.VMEM((2, KBA, BLK), jnp.int32),
            pltpu.VMEM((2, KBA, BLK), jnp.int32),
            pltpu.VMEM((KBA * BLK, D_HID), jnp.float32),
            pltpu.VMEM((BLK, D_HID), jnp.float32),
            pltpu.VMEM_SHARED((NP, D_HID), jnp.float32),
            pltpu.SemaphoreType.DMA,
            pltpu.SemaphoreType.DMA,
            pltpu.SemaphoreType.DMA,
        ],
    )


def _deg_pass(srcp, dstp):
    return _build_deg()(srcp, dstp)


def _agg_pass(srcp, dstp, table):
    return _build_agg()(srcp, dstp, table)


BN = 1024


def _norm(d):
    return lax.rsqrt(jnp.maximum(d, 1.0)).reshape(BN, 1)


def _tc0_body(f_ref, w_ref, o_ref):
    o_ref[...] = jnp.dot(f_ref[...], w_ref[...],
                         preferred_element_type=jnp.float32)


def _tc1_body(h_ref, dg_ref, o_ref):
    ns = _norm(dg_ref[0, 0] + dg_ref[1, 0])
    o_ref[...] = h_ref[...] * ns


def _tc2_body(m_ref, dg_ref, b_ref, o_ref):
    m = m_ref[0] + m_ref[1]
    nd = _norm(dg_ref[0, 1] + dg_ref[1, 1])
    ns = _norm(dg_ref[0, 0] + dg_ref[1, 0])
    x = jnp.maximum(m * nd + b_ref[...], 0.0)
    o_ref[...] = x * ns


def _tc3_body(m_ref, dg_ref, w_ref, b_ref, o_ref):
    m = m_ref[0] + m_ref[1]
    nd = _norm(dg_ref[0, 1] + dg_ref[1, 1])
    o_ref[...] = (jnp.dot(m, w_ref[...], preferred_element_type=jnp.float32)
                  * nd + b_ref[...])[:, :D_OUT]


_deg_spec = pl.BlockSpec((NC, 2, BN), lambda i: (0, 0, i))
_vec_spec = pl.BlockSpec((BN, D_HID), lambda i: (i, 0))
_par_spec = pl.BlockSpec((NC, BN, D_HID), lambda i: (0, i, 0))
_b_spec = pl.BlockSpec((1, D_HID), lambda i: (0, 0))

_tc0_call = pl.pallas_call(
    _tc0_body,
    grid=(NP // BN,),
    in_specs=[pl.BlockSpec((BN, D_IN), lambda i: (i, 0)),
              pl.BlockSpec((D_IN, D_HID), lambda i: (0, 0))],
    out_specs=_vec_spec,
    out_shape=jax.ShapeDtypeStruct((NP, D_HID), jnp.float32),
)

_tc1_call = pl.pallas_call(
    _tc1_body,
    grid=(NP // BN,),
    in_specs=[_vec_spec, _deg_spec],
    out_specs=_vec_spec,
    out_shape=jax.ShapeDtypeStruct((NP, D_HID), jnp.float32),
)

_tc2_call = pl.pallas_call(
    _tc2_body,
    grid=(NP // BN,),
    in_specs=[_par_spec, _deg_spec, _b_spec],
    out_specs=_vec_spec,
    out_shape=jax.ShapeDtypeStruct((NP, D_HID), jnp.float32),
)

_tc3_call = pl.pallas_call(
    _tc3_body,
    grid=(NP // BN,),
    in_specs=[_par_spec, _deg_spec,
              pl.BlockSpec((D_HID, D_HID), lambda i: (0, 0)), _b_spec],
    out_specs=pl.BlockSpec((BN, D_OUT), lambda i: (i, 0)),
    out_shape=jax.ShapeDtypeStruct((N, D_OUT), jnp.float32),
)


def kernel(features, edge_index, W1, b1, W2, b2):
    src = edge_index[0]
    dst = edge_index[1]
    pad = jnp.full((EP - E,), N, dtype=jnp.int32)
    srcp = jnp.concatenate([src, pad]).reshape(NB, BLK)
    dstp = jnp.concatenate([dst, pad]).reshape(NB, BLK)
    W2p = jnp.pad(W2, ((0, 0), (0, D_HID - D_OUT)))
    b1r = b1.reshape(1, D_HID)
    b2p = jnp.pad(b2, (0, D_HID - D_OUT)).reshape(1, D_HID)

    H = _tc0_call(features, W1)
    degs = _deg_pass(srcp, dstp)
    T1 = _tc1_call(H, degs)
    M1 = _agg_pass(srcp, dstp, T1)
    T2 = _tc2_call(M1, degs, b1r)
    M2 = _agg_pass(srcp, dstp, T2)
    return _tc3_call(M2, degs, W2p, b2p)


# speedup vs baseline: 1.2412x; 1.0061x over previous
import functools

import jax
import jax.numpy as jnp
from jax import lax
from jax.experimental import pallas as pl
from jax.experimental.pallas import tpu as pltpu
from jax.experimental.pallas import tpu_sc as plsc

N = 100000
E = 3200000
D_IN = 10
D_HID = 16
D_OUT = 2

NC = 2
NS = 16
NW = NC * NS
BLK = 128
BPW = 800
KB = 16
CHUNKS = BPW // KB
KBA = 10
CHUNKS_A = BPW // KBA
NB = NW * BPW
BPW0_D = 928
BPW1_D = NB // NS - BPW0_D
BPW0_A = 1400
BPW1_A = NB // NS - BPW0_A
EP = NB * BLK
NP = 100352
RPS = NP // NS

def _deg_body(src_h, dst_h, out_h, idx_s, idx_d, ones_v, zb, acc_s, acc_d,
              sem_i, sem_s):
    c = lax.axis_index("c")
    s = lax.axis_index("s")
    wid = c * NS + s

    def fill(i, _):
        zb[pl.ds(i * 16, 16)] = jnp.zeros((16,), jnp.float32)
        return 0

    lax.fori_loop(0, RPS // 16, fill, 0)
    for i in range(BLK // 16):
        ones_v[pl.ds(i * 16, 16)] = jnp.ones((16,), jnp.float32)
    pltpu.sync_copy(zb, acc_s.at[pl.ds(s * RPS, RPS)])
    pltpu.sync_copy(zb, acc_d.at[pl.ds(s * RPS, RPS)])
    plsc.subcore_barrier()

    base = lax.select(c == 0, s * BPW0_D, NS * BPW0_D + s * BPW1_D)
    nchunks = lax.select(c == 0, BPW0_D // KB, BPW1_D // KB)

    def load_idx(k):
        blk0 = base + k * KB
        par = lax.rem(k, 2)
        pltpu.async_copy(src_h.at[pl.ds(blk0, KB)], idx_s.at[par], sem_i)
        pltpu.async_copy(dst_h.at[pl.ds(blk0, KB)], idx_d.at[par], sem_i)

    load_idx(0)

    def chunk(k, _):
        par = lax.rem(k, 2)
        pltpu.make_async_copy(src_h.at[pl.ds(0, KB)], idx_s.at[par], sem_i).wait()
        pltpu.make_async_copy(dst_h.at[pl.ds(0, KB)], idx_d.at[par], sem_i).wait()

        @pl.when(k < nchunks - 1)
        def _():
            load_idx(k + 1)

        for j in range(KB):
            pltpu.sync_copy(ones_v, acc_s.at[idx_s.at[par, j]], add=True)
            pltpu.sync_copy(ones_v, acc_d.at[idx_d.at[par, j]], add=True)
        return 0

    lax.fori_loop(0, nchunks, chunk, 0)
    plsc.subcore_barrier()
    pltpu.sync_copy(acc_s.at[pl.ds(s * RPS, RPS)], out_h.at[c, 0, pl.ds(s * RPS, RPS)])
    pltpu.sync_copy(acc_d.at[pl.ds(s * RPS, RPS)], out_h.at[c, 1, pl.ds(s * RPS, RPS)])


@functools.lru_cache(maxsize=None)
def _build_deg():
    return pl.kernel(
        _deg_body,
        out_type=jax.ShapeDtypeStruct((NC, 2, NP), jnp.float32),
        mesh=plsc.VectorSubcoreMesh(core_axis_name="c", subcore_axis_name="s"),
        compiler_params=pltpu.CompilerParams(
            use_tc_tiling_on_sc=False, skip_device_barrier=True),
        scratch_types=[
            pltpu.VMEM((2, KB, BLK), jnp.int32),
            pltpu.VMEM((2, KB, BLK), jnp.int32),
            pltpu.VMEM((BLK,), jnp.float32),
            pltpu.VMEM((RPS,), jnp.float32),
            pltpu.VMEM_SHARED((NP,), jnp.float32),
            pltpu.VMEM_SHARED((NP,), jnp.float32),
            pltpu.SemaphoreType.DMA,
            pltpu.SemaphoreType.DMA,
        ],
    )


def _agg_body(src_h, dst_h, tab_h, out_h, idx_s, idx_d, rows, zb, acc,
              sem_i, sem_g, sem_s):
    c = lax.axis_index("c")
    s = lax.axis_index("s")
    wid = c * NS + s

    def fillz(i, _):
        zb[i] = jnp.zeros((16,), jnp.float32)
        return 0

    lax.fori_loop(0, BLK, fillz, 0)

    def zero_acc(i, _):
        pltpu.sync_copy(zb, acc.at[pl.ds(s * RPS + i * BLK, BLK)])
        return 0

    lax.fori_loop(0, RPS // BLK, zero_acc, 0)
    plsc.subcore_barrier()

    base = lax.select(c == 0, s * BPW0_A, NS * BPW0_A + s * BPW1_A)
    base = lax.min(base, NB - KBA)
    nchunks = lax.select(c == 0, BPW0_A // KBA, BPW1_A // KBA)

    def load_idx(k):
        blk0 = base + k * KBA
        par = lax.rem(k, 2)
        pltpu.async_copy(src_h.at[pl.ds(blk0, KBA)], idx_s.at[par], sem_i)
        pltpu.async_copy(dst_h.at[pl.ds(blk0, KBA)], idx_d.at[par], sem_i)

    @pl.when(nchunks > 0)
    def _():
        load_idx(0)

    def chunk(k, _):
        par = lax.rem(k, 2)
        @pl.when(k > 0)
        def _():
            pltpu.make_async_copy(
                tab_h.at[pl.ds(0, KBA * BLK)], rows, sem_s).wait()

        pltpu.make_async_copy(src_h.at[pl.ds(0, KBA)], idx_s.at[par], sem_i).wait()
        pltpu.make_async_copy(dst_h.at[pl.ds(0, KBA)], idx_d.at[par], sem_i).wait()

        @pl.when(k < nchunks - 1)
        def _():
            load_idx(k + 1)

        hs = []
        for j in range(KBA):
            hs.append(pltpu.async_copy(
                tab_h.at[idx_s.at[par, j]],
                rows.at[pl.ds(j * BLK, BLK)], sem_g))
        for j in range(KBA):
            hs[j].wait()
            pltpu.async_copy(rows.at[pl.ds(j * BLK, BLK)],
                             acc.at[idx_d.at[par, j]], sem_s, add=True)
        return 0

    lax.fori_loop(0, nchunks, chunk, 0)

    @pl.when(nchunks > 0)
    def _():
        pltpu.make_async_copy(tab_h.at[pl.ds(0, KBA * BLK)], rows, sem_s).wait()

    plsc.subcore_barrier()
    pltpu.sync_copy(acc.at[pl.ds(s * RPS, RPS)], out_h.at[c, pl.ds(s * RPS, RPS)])


@functools.lru_cache(maxsize=None)
def _build_agg():
    return pl.kernel(
        _agg_body,
        out_type=jax.ShapeDtypeStruct((NC, NP, D_HID), jnp.float32),
        mesh=plsc.VectorSubcoreMesh(core_axis_name="c", subcore_axis_name="s"),
        compiler_params=pltpu.CompilerParams(
            use_tc_tiling_on_sc=False, skip_device_barrier=True),
        scratch_types=[
            pltpu.VMEM((2, KBA, BLK), jnp.int32),
            pltpu.VMEM((2, KBA, BLK), jnp.int32),
            pltpu.VMEM((KBA * BLK, D_HID), jnp.float32),
            pltpu.VMEM((BLK, D_HID), jnp.float32),
            pltpu.VMEM_SHARED((NP, D_HID), jnp.float32),
            pltpu.SemaphoreType.DMA,
            pltpu.SemaphoreType.DMA,
            pltpu.SemaphoreType.DMA,
        ],
    )


def _deg_pass(srcp, dstp):
    return _build_deg()(srcp, dstp)


def _agg_pass(srcp, dstp, table):
    return _build_agg()(srcp, dstp, table)


BN = 1024


def _norm(d):
    return lax.rsqrt(jnp.maximum(d, 1.0)).reshape(BN, 1)


def _tc0_body(f_ref, w_ref, o_ref):
    o_ref[...] = jnp.dot(f_ref[...], w_ref[...],
                         preferred_element_type=jnp.float32)


def _tc1_body(h_ref, dg_ref, o_ref):
    ns = _norm(dg_ref[0, 0] + dg_ref[1, 0])
    o_ref[...] = h_ref[...] * ns


def _tc2_body(m_ref, dg_ref, b_ref, o_ref):
    m = m_ref[0] + m_ref[1]
    nd = _norm(dg_ref[0, 1] + dg_ref[1, 1])
    ns = _norm(dg_ref[0, 0] + dg_ref[1, 0])
    x = jnp.maximum(m * nd + b_ref[...], 0.0)
    o_ref[...] = x * ns


def _tc3_body(m_ref, dg_ref, w_ref, b_ref, o_ref):
    m = m_ref[0] + m_ref[1]
    nd = _norm(dg_ref[0, 1] + dg_ref[1, 1])
    o_ref[...] = (jnp.dot(m, w_ref[...], preferred_element_type=jnp.float32)
                  * nd + b_ref[...])[:, :D_OUT]


_deg_spec = pl.BlockSpec((NC, 2, BN), lambda i: (0, 0, i))
_vec_spec = pl.BlockSpec((BN, D_HID), lambda i: (i, 0))
_par_spec = pl.BlockSpec((NC, BN, D_HID), lambda i: (0, i, 0))
_b_spec = pl.BlockSpec((1, D_HID), lambda i: (0, 0))

_tc0_call = pl.pallas_call(
    _tc0_body,
    grid=(NP // BN,),
    in_specs=[pl.BlockSpec((BN, D_IN), lambda i: (i, 0)),
              pl.BlockSpec((D_IN, D_HID), lambda i: (0, 0))],
    out_specs=_vec_spec,
    out_shape=jax.ShapeDtypeStruct((NP, D_HID), jnp.float32),
)

_tc1_call = pl.pallas_call(
    _tc1_body,
    grid=(NP // BN,),
    in_specs=[_vec_spec, _deg_spec],
    out_specs=_vec_spec,
    out_shape=jax.ShapeDtypeStruct((NP, D_HID), jnp.float32),
)

_tc2_call = pl.pallas_call(
    _tc2_body,
    grid=(NP // BN,),
    in_specs=[_par_spec, _deg_spec, _b_spec],
    out_specs=_vec_spec,
    out_shape=jax.ShapeDtypeStruct((NP, D_HID), jnp.float32),
)

_tc3_call = pl.pallas_call(
    _tc3_body,
    grid=(NP // BN,),
    in_specs=[_par_spec, _deg_spec,
              pl.BlockSpec((D_HID, D_HID), lambda i: (0, 0)), _b_spec],
    out_specs=pl.BlockSpec((BN, D_OUT), lambda i: (i, 0)),
    out_shape=jax.ShapeDtypeStruct((N, D_OUT), jnp.float32),
)


def kernel(features, edge_index, W1, b1, W2, b2):
    src = edge_index[0]
    dst = edge_index[1]
    pad = jnp.full((EP - E,), N, dtype=jnp.int32)
    srcp = jnp.concatenate([src, pad]).reshape(NB, BLK)
    dstp = jnp.concatenate([dst, pad]).reshape(NB, BLK)
    W2p = jnp.pad(W2, ((0, 0), (0, D_HID - D_OUT)))
    b1r = b1.reshape(1, D_HID)
    b2p = jnp.pad(b2, (0, D_HID - D_OUT)).reshape(1, D_HID)

    H = _tc0_call(features, W1)
    degs = _deg_pass(srcp, dstp)
    T1 = _tc1_call(H, degs)
    M1 = _agg_pass(srcp, dstp, T1)
    T2 = _tc2_call(M1, degs, b1r)
    M2 = _agg_pass(srcp, dstp, T2)
    return _tc3_call(M2, degs, W2p, b2p)
